# baseline (speedup 1.0000x reference)
import jax
import jax.numpy as jnp
from jax.experimental import pallas as pl
from jax.experimental.pallas import tpu as pltpu

B, S, D, H = 1, 2048, 1024, 16
E, K, HID = 4, 2, 4096
DH = D // H
BT = 256
HB = 512
NT = S // BT
NHB = HID // HB

_f32 = jnp.float32
_bf16 = jnp.bfloat16


def _dot(a, b, trans_b=False, prec=None):
    dims = (((1,), (1 if trans_b else 0,)), ((), ()))
    return jax.lax.dot_general(a, b, dims, preferred_element_type=_f32,
                               precision=prec)


_HI = jax.lax.Precision.HIGHEST


def _qkv_kern(x_ref, g_ref, b_ref, w_ref, bias_ref, o_ref):
    x = x_ref[...]
    m = jnp.mean(x, axis=-1, keepdims=True)
    v = jnp.mean((x - m) ** 2, axis=-1, keepdims=True)
    h = (x - m) / jnp.sqrt(v + 1e-5) * g_ref[...] + b_ref[...]
    o_ref[...] = _dot(h.astype(_bf16), w_ref[...]) + bias_ref[...]


def _attn_kern(q_ref, k_ref, v_ref, o_ref, s_scr):
    t = pl.program_id(1)
    q = q_ref[0].astype(_bf16)

    def p1(kb, m):
        kblk = k_ref[0, pl.ds(kb * BT, BT), :].astype(_bf16)
        s = _dot(q, kblk, trans_b=True) * (1.0 / 8.0)
        row = t * BT + jax.lax.broadcasted_iota(jnp.int32, (BT, BT), 0)
        col = kb * BT + jax.lax.broadcasted_iota(jnp.int32, (BT, BT), 1)
        s = jnp.where(col <= row, s, -1e9)
        s_scr[:, pl.ds(kb * BT, BT)] = s
        return jnp.maximum(m, jnp.max(s, axis=-1, keepdims=True))

    m = jax.lax.fori_loop(0, t + 1, p1, jnp.full((BT, 1), -1e30, _f32))

    def p2(kb, l):
        e = jnp.exp(s_scr[:, pl.ds(kb * BT, BT)] - m)
        s_scr[:, pl.ds(kb * BT, BT)] = e
        return l + jnp.sum(e, axis=-1, keepdims=True)

    l = jax.lax.fori_loop(0, t + 1, p2, jnp.zeros((BT, 1), _f32))

    def p3(kb, acc):
        p = (s_scr[:, pl.ds(kb * BT, BT)] / l).astype(_bf16)
        vblk = v_ref[0, pl.ds(kb * BT, BT), :].astype(_bf16)
        return acc + _dot(p, vblk)

    o_ref[0] = jax.lax.fori_loop(0, t + 1, p3, jnp.zeros((BT, DH), _f32))


def _post_kern(ctx_ref, wo_ref, bo_ref, x_ref, g_ref, b_ref, wg_ref,
               x1_ref, tok_ref, lg_ref):
    ao = _dot(ctx_ref[...].astype(_bf16), wo_ref[...]) + bo_ref[...]
    x1 = x_ref[...] + ao
    x1_ref[...] = x1
    m = jnp.mean(x1, axis=-1, keepdims=True)
    v = jnp.mean((x1 - m) ** 2, axis=-1, keepdims=True)
    tok = (x1 - m) / jnp.sqrt(v + 1e-5) * g_ref[...] + b_ref[...]
    tok_ref[...] = tok
    lg_ref[...] = _dot(tok.astype(_bf16), wg_ref[...].astype(_bf16))


def _route_kern(lg_ref, w_ref, aux_ref):
    lg = lg_ref[...]
    m = jnp.max(lg, axis=-1, keepdims=True)
    ex = jnp.exp(lg - m)
    p = ex / jnp.sum(ex, axis=-1, keepdims=True)
    iota = jax.lax.broadcasted_iota(jnp.int32, (S, E), 1)
    m1 = jnp.max(p, axis=-1, keepdims=True)
    i1 = jnp.min(jnp.where(p == m1, iota, E), axis=-1, keepdims=True)
    pm = jnp.where(iota == i1, -1.0, p)
    m2 = jnp.max(pm, axis=-1, keepdims=True)
    i2 = jnp.min(jnp.where(pm == m2, iota, E), axis=-1, keepdims=True)
    wsum = m1 + m2
    w = jnp.where(iota == i1, m1 / wsum, 0.0) + jnp.where(iota == i2, m2 / wsum, 0.0)
    w_ref[...] = w
    sel = jnp.logical_or(iota == i1, iota == i2)
    fi = jnp.sum(sel.astype(_f32), axis=0, keepdims=True) / (S * K)
    Pi = jnp.mean(p, axis=0, keepdims=True)
    aux_ref[...] = 0.01 * E * jnp.sum(fi * Pi, axis=-1, keepdims=True)


def _ffn_kern(tok_ref, x1_ref, w_ref, we1_ref, be1_ref, we2_ref, be2_ref,
              ws1_ref, bs1_ref, ws2_ref, bs2_ref, o_ref):
    hb = pl.program_id(0)
    t = pl.program_id(1)
    rows = pl.ds(t * BT, BT)
    tokb = tok_ref[rows, :].astype(_bf16)
    eh = jax.nn.gelu(_dot(tokb, ws1_ref[...]) + bs1_ref[...])
    acc = _dot(eh.astype(_bf16), ws2_ref[...])
    for e in range(E):
        ehe = jax.nn.gelu(_dot(tokb, we1_ref[e]) + be1_ref[e:e + 1, :])
        pe = _dot(ehe.astype(_bf16), we2_ref[e])
        acc += w_ref[rows, e:e + 1] * pe

    @pl.when(hb == 0)
    def _():
        base = x1_ref[rows, :] + bs2_ref[...]
        for e in range(E):
            base += w_ref[rows, e:e + 1] * be2_ref[e:e + 1, :]
        o_ref[rows, :] = base + acc

    @pl.when(hb != 0)
    def _():
        o_ref[rows, :] += acc


def kernel(x, ln1_g, ln1_b, Wqkv, bqkv, Wo, bo, ln2_g, ln2_b, Wg,
           We1, be1, We2, be2, Ws1, bs1, Ws2, bs2):
    x2 = x.reshape(S, D)
    row1 = lambda a: a.reshape(1, -1)

    qkv = pl.pallas_call(
        _qkv_kern,
        grid=(NT,),
        in_specs=[
            pl.BlockSpec((BT, D), lambda i: (i, 0)),
            pl.BlockSpec((1, D), lambda i: (0, 0)),
            pl.BlockSpec((1, D), lambda i: (0, 0)),
            pl.BlockSpec((D, 3 * D), lambda i: (0, 0)),
            pl.BlockSpec((1, 3 * D), lambda i: (0, 0)),
        ],
        out_specs=pl.BlockSpec((BT, 3 * D), lambda i: (i, 0)),
        out_shape=jax.ShapeDtypeStruct((S, 3 * D), _f32),
    )(x2, row1(ln1_g), row1(ln1_b), Wqkv.astype(_bf16), row1(bqkv))

    q3 = qkv[:, :D].reshape(S, H, DH).transpose(1, 0, 2)
    k3 = qkv[:, D:2 * D].reshape(S, H, DH).transpose(1, 0, 2)
    v3 = qkv[:, 2 * D:].reshape(S, H, DH).transpose(1, 0, 2)

    ctx3 = pl.pallas_call(
        _attn_kern,
        grid=(H, NT),
        in_specs=[
            pl.BlockSpec((1, BT, DH), lambda h, t: (h, t, 0)),
            pl.BlockSpec((1, S, DH), lambda h, t: (h, 0, 0)),
            pl.BlockSpec((1, S, DH), lambda h, t: (h, 0, 0)),
        ],
        out_specs=pl.BlockSpec((1, BT, DH), lambda h, t: (h, t, 0)),
        out_shape=jax.ShapeDtypeStruct((H, S, DH), _f32),
        scratch_shapes=[pltpu.VMEM((BT, S), _f32)],
    )(q3, k3, v3)
    ctx = ctx3.transpose(1, 0, 2).reshape(S, D)

    x1, tok, logits = pl.pallas_call(
        _post_kern,
        grid=(NT,),
        in_specs=[
            pl.BlockSpec((BT, D), lambda i: (i, 0)),
            pl.BlockSpec((D, D), lambda i: (0, 0)),
            pl.BlockSpec((1, D), lambda i: (0, 0)),
            pl.BlockSpec((BT, D), lambda i: (i, 0)),
            pl.BlockSpec((1, D), lambda i: (0, 0)),
            pl.BlockSpec((1, D), lambda i: (0, 0)),
            pl.BlockSpec((D, E), lambda i: (0, 0)),
        ],
        out_specs=[
            pl.BlockSpec((BT, D), lambda i: (i, 0)),
            pl.BlockSpec((BT, D), lambda i: (i, 0)),
            pl.BlockSpec((BT, E), lambda i: (i, 0)),
        ],
        out_shape=[
            jax.ShapeDtypeStruct((S, D), _f32),
            jax.ShapeDtypeStruct((S, D), _f32),
            jax.ShapeDtypeStruct((S, E), _f32),
        ],
    )(ctx, Wo.astype(_bf16), row1(bo), x2, row1(ln2_g), row1(ln2_b), Wg)

    w, aux = pl.pallas_call(
        _route_kern,
        grid=(1,),
        in_specs=[pl.BlockSpec((S, E), lambda i: (0, 0))],
        out_specs=[
            pl.BlockSpec((S, E), lambda i: (0, 0)),
            pl.BlockSpec((1, 1), lambda i: (0, 0)),
        ],
        out_shape=[
            jax.ShapeDtypeStruct((S, E), _f32),
            jax.ShapeDtypeStruct((1, 1), _f32),
        ],
    )(logits)

    out = pl.pallas_call(
        _ffn_kern,
        grid=(NHB, NT),
        in_specs=[
            pl.BlockSpec((S, D), lambda hb, t: (0, 0)),
            pl.BlockSpec((S, D), lambda hb, t: (0, 0)),
            pl.BlockSpec((S, E), lambda hb, t: (0, 0)),
            pl.BlockSpec((E, D, HB), lambda hb, t: (0, 0, hb)),
            pl.BlockSpec((E, HB), lambda hb, t: (0, hb)),
            pl.BlockSpec((E, HB, D), lambda hb, t: (0, hb, 0)),
            pl.BlockSpec((E, D), lambda hb, t: (0, 0)),
            pl.BlockSpec((D, HB), lambda hb, t: (0, hb)),
            pl.BlockSpec((1, HB), lambda hb, t: (0, hb)),
            pl.BlockSpec((HB, D), lambda hb, t: (hb, 0)),
            pl.BlockSpec((1, D), lambda hb, t: (0, 0)),
        ],
        out_specs=pl.BlockSpec((S, D), lambda hb, t: (0, 0)),
        out_shape=jax.ShapeDtypeStruct((S, D), _f32),
    )(tok, x1, w, We1.astype(_bf16), be1, We2.astype(_bf16), be2,
      Ws1.astype(_bf16), row1(bs1), Ws2.astype(_bf16), row1(bs2))

    return (aux[0, 0], out.reshape(B, S, D))


# speedup vs baseline: 1.0799x; 1.0799x over previous
import jax
import jax.numpy as jnp
from jax.experimental import pallas as pl
from jax.experimental.pallas import tpu as pltpu
from jax.experimental.pallas import tpu_sc as plsc

B, S, D, H = 1, 2048, 1024, 16
E, K, HID = 4, 2, 4096
DH = D // H
BT = 256
NT = S // BT
HB = 512
NHB = HID // HB
TEX = 512
GMAX = 11
P = GMAX * TEX
CH = 512
LW = 128
NSUB = D // LW
SCW = 128

_f32 = jnp.float32
_bf16 = jnp.bfloat16


def _dot(a, b, trans_b=False, prec=None):
    dims = (((1,), (1 if trans_b else 0,)), ((), ()))
    return jax.lax.dot_general(a, b, dims, preferred_element_type=_f32,
                               precision=prec)


def _qkv_kern(x_ref, g_ref, b_ref, w_ref, bias_ref, o_ref):
    x = x_ref[...]
    m = jnp.mean(x, axis=-1, keepdims=True)
    v = jnp.mean((x - m) ** 2, axis=-1, keepdims=True)
    h = (x - m) / jnp.sqrt(v + 1e-5) * g_ref[...] + b_ref[...]
    o_ref[...] = _dot(h.astype(_bf16), w_ref[...]) + bias_ref[...]


def _attn_kern(q_ref, k_ref, v_ref, o_ref, s_scr):
    t = pl.program_id(1)
    q = q_ref[0].astype(_bf16)

    def fill(kb, c):
        @pl.when(kb <= t)
        def _():
            kblk = k_ref[0, pl.ds(kb * BT, BT), :].astype(_bf16)
            s = _dot(q, kblk, trans_b=True) * (1.0 / 8.0)
            row = t * BT + jax.lax.broadcasted_iota(jnp.int32, (BT, BT), 0)
            col = kb * BT + jax.lax.broadcasted_iota(jnp.int32, (BT, BT), 1)
            s_scr[:, pl.ds(kb * BT, BT)] = jnp.where(col <= row, s, -1e9)

        @pl.when(kb > t)
        def _():
            s_scr[:, pl.ds(kb * BT, BT)] = jnp.full((BT, BT), -1e9, _f32)

        return c

    jax.lax.fori_loop(0, NT, fill, 0)
    s = s_scr[...]
    m = jnp.max(s, axis=-1, keepdims=True)
    e = jnp.exp(s - m)
    p = e / jnp.sum(e, axis=-1, keepdims=True)
    o_ref[0] = _dot(p.astype(_bf16), v_ref[0].astype(_bf16))


def _post_kern(ctx_ref, wo_ref, bo_ref, x_ref, g_ref, b_ref, wg_ref,
               x1_ref, tok_ref, tokb_ref, lg_ref):
    ao = _dot(ctx_ref[...].astype(_bf16), wo_ref[...]) + bo_ref[...]
    x1 = x_ref[...] + ao
    x1_ref[...] = x1
    m = jnp.mean(x1, axis=-1, keepdims=True)
    v = jnp.mean((x1 - m) ** 2, axis=-1, keepdims=True)
    tok = (x1 - m) / jnp.sqrt(v + 1e-5) * g_ref[...] + b_ref[...]
    tok_ref[...] = tok
    tokb = tok.astype(_bf16)
    tokb_ref[...] = tokb
    lg_ref[...] = _dot(tokb, wg_ref[...].astype(_bf16))


def _route_kern(lg_ref, wa_ref, wb_ref, pa_ref, pb_ref, ext_ref, used_ref,
                aux_ref):
    lg = lg_ref[...]
    mx = jnp.max(lg, axis=-1, keepdims=True)
    exl = jnp.exp(lg - mx)
    p = exl / jnp.sum(exl, axis=-1, keepdims=True)
    iota = jax.lax.broadcasted_iota(jnp.int32, (S, E), 1)
    m1 = jnp.max(p, axis=-1, keepdims=True)
    i1 = jnp.min(jnp.where(p == m1, iota, E), axis=-1, keepdims=True)
    pm = jnp.where(iota == i1, -1.0, p)
    m2 = jnp.max(pm, axis=-1, keepdims=True)
    i2 = jnp.min(jnp.where(pm == m2, iota, E), axis=-1, keepdims=True)
    wsum = m1 + m2
    wa_ref[...] = m1 / wsum
    wb_ref[...] = m2 / wsum

    ohA = (iota == i1).astype(_f32)
    ohB = (iota == i2).astype(_f32)
    oh = ohA + ohB

    r = jax.lax.broadcasted_iota(jnp.int32, (CH, CH), 0)
    c = jax.lax.broadcasted_iota(jnp.int32, (CH, CH), 1)
    tril = (r > c).astype(_bf16)
    carry = jnp.zeros((1, E), _f32)
    parts = []
    for ci in range(S // CH):
        blk = oh[ci * CH:(ci + 1) * CH]
        parts.append(_dot(tril, blk.astype(_bf16)) + carry)
        carry = carry + jnp.sum(blk, axis=0, keepdims=True)
    C = jnp.concatenate(parts, axis=0)
    counts = carry

    padded = jnp.floor((counts + (TEX - 1)) * (1.0 / TEX)) * TEX
    er = jax.lax.broadcasted_iota(jnp.int32, (E, E), 0)
    ec = jax.lax.broadcasted_iota(jnp.int32, (E, E), 1)
    off = _dot(padded.astype(_bf16), (er < ec).astype(_bf16))
    end_tile = (off + padded) * (1.0 / TEX)

    posA = jnp.sum(ohA * (off + C), axis=-1, keepdims=True)
    posB = jnp.sum(ohB * (off + C), axis=-1, keepdims=True)
    sub = jax.lax.broadcasted_iota(jnp.int32, (S, NSUB), 1)
    pa_ref[...] = posA.astype(jnp.int32) * NSUB + sub
    pb_ref[...] = posB.astype(jnp.int32) * NSUB + sub

    gi = jax.lax.broadcasted_iota(jnp.int32, (1, GMAX), 1).astype(_f32)
    ext = jnp.zeros((1, GMAX), _f32)
    for e in range(E):
        ext = ext + (gi >= end_tile[:, e:e + 1]).astype(_f32)
    ext_ref[...] = jnp.minimum(ext, E - 1).astype(jnp.int32)
    used_ref[...] = end_tile[:, E - 1:E].astype(jnp.int32)

    sel = jnp.logical_or(iota == i1, iota == i2)
    fi = jnp.sum(sel.astype(_f32), axis=0, keepdims=True) / (S * K)
    Pi = jnp.mean(p, axis=0, keepdims=True)
    aux_ref[...] = 0.01 * E * jnp.sum(fi * Pi, axis=-1, keepdims=True)


def _vmesh():
    return plsc.VectorSubcoreMesh(core_axis_name="c", subcore_axis_name="s")


def _sc_dispatch(tok8, pa2, pb2):
    @pl.kernel(out_type=jax.ShapeDtypeStruct((P * NSUB, LW), _f32),
               mesh=_vmesh(), scratch_types=[])
    def k(tok_hbm, ia_hbm, ib_hbm, o_hbm):
        def scat(x_vmem, i_vmem):
            pltpu.sync_copy(x_vmem, o_hbm.at[i_vmem.at[0]])

        for idx_hbm in (ia_hbm, ib_hbm):
            pltpu.emit_pipeline(
                scat,
                grid=(S * NSUB // SCW,),
                in_specs=[pl.BlockSpec((SCW, LW), index_map=lambda i: (i, 0)),
                          pl.BlockSpec((1, SCW), index_map=lambda i: (0, i))],
                out_specs=[],
                core_axis_name=("c", "s"),
                dimension_semantics=(pltpu.PARALLEL,),
            )(tok_hbm, idx_hbm)

    return k(tok8, pa2, pb2)


def _sc_combine(eo8, pa2, pb2):
    @pl.kernel(out_type=(jax.ShapeDtypeStruct((S * NSUB, LW), _f32),
                         jax.ShapeDtypeStruct((S * NSUB, LW), _f32)),
               mesh=_vmesh(), scratch_types=[])
    def k(eo_hbm, ia_hbm, ib_hbm, ga_hbm, gb_hbm):
        def gath(i_vmem, o_vmem):
            pltpu.sync_copy(eo_hbm.at[i_vmem.at[0]], o_vmem)

        for idx_hbm, out_hbm in ((ia_hbm, ga_hbm), (ib_hbm, gb_hbm)):
            pltpu.emit_pipeline(
                gath,
                grid=(S * NSUB // SCW,),
                in_specs=[pl.BlockSpec((1, SCW), index_map=lambda i: (0, i))],
                out_specs=[pl.BlockSpec((SCW, LW), index_map=lambda i: (i, 0))],
                core_axis_name=("c", "s"),
                dimension_semantics=(pltpu.PARALLEL,),
            )(idx_hbm, out_hbm)

    return k(eo8, pa2, pb2)


def _gmm_kern(ext_ref, used_ref, x_ref, we1_ref, be1_ref, we2_ref, be2_ref,
              o_ref):
    g = pl.program_id(0)

    @pl.when(g < used_ref[0])
    def _():
        xb = x_ref[...].astype(_bf16)
        eh = jax.nn.gelu(_dot(xb, we1_ref[0]) + be1_ref[0])
        o_ref[...] = _dot(eh.astype(_bf16), we2_ref[0]) + be2_ref[0]


def _shared_kern(tok_ref, bs2_ref, ws1_ref, bs1_ref, ws2_ref, o_ref):
    hb = pl.program_id(0)
    t = pl.program_id(1)
    rows = pl.ds(t * BT, BT)
    tokb = tok_ref[rows, :]
    eh = jax.nn.gelu(_dot(tokb, ws1_ref[...]) + bs1_ref[...])
    acc = _dot(eh.astype(_bf16), ws2_ref[...])

    @pl.when(hb == 0)
    def _():
        o_ref[rows, :] = acc + bs2_ref[...]

    @pl.when(hb != 0)
    def _():
        o_ref[rows, :] += acc


def _comb_kern(x1_ref, sh_ref, wa_ref, wb_ref, ga_ref, gb_ref, o_ref):
    routed = wa_ref[...] * ga_ref[...] + wb_ref[...] * gb_ref[...]
    o_ref[...] = x1_ref[...] + sh_ref[...] + routed


def kernel(x, ln1_g, ln1_b, Wqkv, bqkv, Wo, bo, ln2_g, ln2_b, Wg,
           We1, be1, We2, be2, Ws1, bs1, Ws2, bs2):
    x2 = x.reshape(S, D)
    row1 = lambda a: a.reshape(1, -1)

    qkv = pl.pallas_call(
        _qkv_kern,
        grid=(NT,),
        in_specs=[
            pl.BlockSpec((BT, D), lambda i: (i, 0)),
            pl.BlockSpec((1, D), lambda i: (0, 0)),
            pl.BlockSpec((1, D), lambda i: (0, 0)),
            pl.BlockSpec((D, 3 * D), lambda i: (0, 0)),
            pl.BlockSpec((1, 3 * D), lambda i: (0, 0)),
        ],
        out_specs=pl.BlockSpec((BT, 3 * D), lambda i: (i, 0)),
        out_shape=jax.ShapeDtypeStruct((S, 3 * D), _f32),
    )(x2, row1(ln1_g), row1(ln1_b), Wqkv.astype(_bf16), row1(bqkv))

    q3 = qkv[:, :D].reshape(S, H, DH).transpose(1, 0, 2)
    k3 = qkv[:, D:2 * D].reshape(S, H, DH).transpose(1, 0, 2)
    v3 = qkv[:, 2 * D:].reshape(S, H, DH).transpose(1, 0, 2)

    ctx3 = pl.pallas_call(
        _attn_kern,
        grid=(H, NT),
        in_specs=[
            pl.BlockSpec((1, BT, DH), lambda h, t: (h, t, 0)),
            pl.BlockSpec((1, S, DH), lambda h, t: (h, 0, 0)),
            pl.BlockSpec((1, S, DH), lambda h, t: (h, 0, 0)),
        ],
        out_specs=pl.BlockSpec((1, BT, DH), lambda h, t: (h, t, 0)),
        out_shape=jax.ShapeDtypeStruct((H, S, DH), _f32),
        scratch_shapes=[pltpu.VMEM((BT, S), _f32)],
    )(q3, k3, v3)
    ctx = ctx3.transpose(1, 0, 2).reshape(S, D)

    x1, tok, tokb, logits = pl.pallas_call(
        _post_kern,
        grid=(NT,),
        in_specs=[
            pl.BlockSpec((BT, D), lambda i: (i, 0)),
            pl.BlockSpec((D, D), lambda i: (0, 0)),
            pl.BlockSpec((1, D), lambda i: (0, 0)),
            pl.BlockSpec((BT, D), lambda i: (i, 0)),
            pl.BlockSpec((1, D), lambda i: (0, 0)),
            pl.BlockSpec((1, D), lambda i: (0, 0)),
            pl.BlockSpec((D, E), lambda i: (0, 0)),
        ],
        out_specs=[
            pl.BlockSpec((BT, D), lambda i: (i, 0)),
            pl.BlockSpec((BT, D), lambda i: (i, 0)),
            pl.BlockSpec((BT, D), lambda i: (i, 0)),
            pl.BlockSpec((BT, E), lambda i: (i, 0)),
        ],
        out_shape=[
            jax.ShapeDtypeStruct((S, D), _f32),
            jax.ShapeDtypeStruct((S, D), _f32),
            jax.ShapeDtypeStruct((S, D), _bf16),
            jax.ShapeDtypeStruct((S, E), _f32),
        ],
    )(ctx, Wo.astype(_bf16), row1(bo), x2, row1(ln2_g), row1(ln2_b), Wg)

    wa, wb, pa, pb, ext, used, aux = pl.pallas_call(
        _route_kern,
        grid=(1,),
        in_specs=[pl.BlockSpec((S, E), lambda i: (0, 0))],
        out_specs=[
            pl.BlockSpec((S, 1), lambda i: (0, 0)),
            pl.BlockSpec((S, 1), lambda i: (0, 0)),
            pl.BlockSpec((S, NSUB), lambda i: (0, 0)),
            pl.BlockSpec((S, NSUB), lambda i: (0, 0)),
            pl.BlockSpec((1, GMAX), lambda i: (0, 0)),
            pl.BlockSpec((1, 1), lambda i: (0, 0)),
            pl.BlockSpec((1, 1), lambda i: (0, 0)),
        ],
        out_shape=[
            jax.ShapeDtypeStruct((S, 1), _f32),
            jax.ShapeDtypeStruct((S, 1), _f32),
            jax.ShapeDtypeStruct((S, NSUB), jnp.int32),
            jax.ShapeDtypeStruct((S, NSUB), jnp.int32),
            jax.ShapeDtypeStruct((1, GMAX), jnp.int32),
            jax.ShapeDtypeStruct((1, 1), jnp.int32),
            jax.ShapeDtypeStruct((1, 1), _f32),
        ],
    )(logits)

    pa2 = pa.reshape(1, S * NSUB)
    pb2 = pb.reshape(1, S * NSUB)

    xs = _sc_dispatch(tok.reshape(S * NSUB, LW), pa2, pb2).reshape(P, D)

    eo = pl.pallas_call(
        _gmm_kern,
        grid_spec=pltpu.PrefetchScalarGridSpec(
            num_scalar_prefetch=2,
            grid=(GMAX,),
            in_specs=[
                pl.BlockSpec((TEX, D), lambda g, ext, used: (g, 0)),
                pl.BlockSpec((1, D, HID), lambda g, ext, used: (ext[g], 0, 0)),
                pl.BlockSpec((1, 1, HID), lambda g, ext, used: (ext[g], 0, 0)),
                pl.BlockSpec((1, HID, D), lambda g, ext, used: (ext[g], 0, 0)),
                pl.BlockSpec((1, 1, D), lambda g, ext, used: (ext[g], 0, 0)),
            ],
            out_specs=pl.BlockSpec((TEX, D), lambda g, ext, used: (g, 0)),
        ),
        out_shape=jax.ShapeDtypeStruct((P, D), _f32),
    )(ext.reshape(GMAX), used.reshape(1), xs, We1.astype(_bf16),
      be1.reshape(E, 1, HID), We2.astype(_bf16), be2.reshape(E, 1, D))

    ga, gb = _sc_combine(eo.reshape(P * NSUB, LW), pa2, pb2)
    ga = ga.reshape(S, D)
    gb = gb.reshape(S, D)

    shared = pl.pallas_call(
        _shared_kern,
        grid=(NHB, NT),
        in_specs=[
            pl.BlockSpec((S, D), lambda hb, t: (0, 0)),
            pl.BlockSpec((1, D), lambda hb, t: (0, 0)),
            pl.BlockSpec((D, HB), lambda hb, t: (0, hb)),
            pl.BlockSpec((1, HB), lambda hb, t: (0, hb)),
            pl.BlockSpec((HB, D), lambda hb, t: (hb, 0)),
        ],
        out_specs=pl.BlockSpec((S, D), lambda hb, t: (0, 0)),
        out_shape=jax.ShapeDtypeStruct((S, D), _f32),
    )(tokb, row1(bs2), Ws1.astype(_bf16), row1(bs1), Ws2.astype(_bf16))

    out = pl.pallas_call(
        _comb_kern,
        grid=(NT,),
        in_specs=[
            pl.BlockSpec((BT, D), lambda i: (i, 0)),
            pl.BlockSpec((BT, D), lambda i: (i, 0)),
            pl.BlockSpec((BT, 1), lambda i: (i, 0)),
            pl.BlockSpec((BT, 1), lambda i: (i, 0)),
            pl.BlockSpec((BT, D), lambda i: (i, 0)),
            pl.BlockSpec((BT, D), lambda i: (i, 0)),
        ],
        out_specs=pl.BlockSpec((BT, D), lambda i: (i, 0)),
        out_shape=jax.ShapeDtypeStruct((S, D), _f32),
    )(x1, shared, wa, wb, ga, gb)

    return (aux[0, 0], out.reshape(B, S, D))


# speedup vs baseline: 1.2250x; 1.1343x over previous
import jax
import jax.numpy as jnp
from jax.experimental import pallas as pl
from jax.experimental.pallas import tpu as pltpu
from jax.experimental.pallas import tpu_sc as plsc

B, S, D, H = 1, 2048, 1024, 16
E, K, HID = 4, 2, 4096
DH = D // H
BT = 256
NT = S // BT
BQ = 512
NQ = S // BQ
HB = 512
NHB = HID // HB
TEX = 512
GMAX = 11
P = GMAX * TEX
CH = 512
LW = 128
NSUB = D // LW
SCW = 128

_f32 = jnp.float32
_bf16 = jnp.bfloat16


def _dot(a, b, trans_b=False, prec=None):
    dims = (((1,), (1 if trans_b else 0,)), ((), ()))
    return jax.lax.dot_general(a, b, dims, preferred_element_type=_f32,
                               precision=prec)


def _qkv_kern(x_ref, g_ref, b_ref, w_ref, bias_ref, o_ref):
    x = x_ref[...]
    m = jnp.mean(x, axis=-1, keepdims=True)
    v = jnp.mean((x - m) ** 2, axis=-1, keepdims=True)
    h = (x - m) / jnp.sqrt(v + 1e-5) * g_ref[...] + b_ref[...]
    o_ref[...] = _dot(h.astype(_bf16), w_ref[...]) + bias_ref[...]


def _attn_kern(q_ref, k_ref, v_ref, o_ref, s_scr):
    t = pl.program_id(1)
    q = q_ref[0].astype(_bf16)

    def fill(kb, c):
        @pl.when(kb <= t)
        def _():
            kblk = k_ref[0, pl.ds(kb * BQ, BQ), :].astype(_bf16)
            s = _dot(q, kblk, trans_b=True) * (1.0 / 8.0)
            row = t * BQ + jax.lax.broadcasted_iota(jnp.int32, (BQ, BQ), 0)
            col = kb * BQ + jax.lax.broadcasted_iota(jnp.int32, (BQ, BQ), 1)
            s_scr[:, pl.ds(kb * BQ, BQ)] = jnp.where(col <= row, s, -1e9)

        @pl.when(kb > t)
        def _():
            s_scr[:, pl.ds(kb * BQ, BQ)] = jnp.full((BQ, BQ), -1e9, _f32)

        return c

    jax.lax.fori_loop(0, NQ, fill, 0)
    s = s_scr[...]
    m = jnp.max(s, axis=-1, keepdims=True)
    e = jnp.exp(s - m)
    p = e / jnp.sum(e, axis=-1, keepdims=True)
    o_ref[0] = _dot(p.astype(_bf16), v_ref[0].astype(_bf16))


def _post_kern(ctx_ref, wo_ref, bo_ref, x_ref, g_ref, b_ref, wg_ref,
               x1_ref, tok_ref, tokb_ref, lg_ref):
    ao = _dot(ctx_ref[...].astype(_bf16), wo_ref[...]) + bo_ref[...]
    x1 = x_ref[...] + ao
    x1_ref[...] = x1
    m = jnp.mean(x1, axis=-1, keepdims=True)
    v = jnp.mean((x1 - m) ** 2, axis=-1, keepdims=True)
    tok = (x1 - m) / jnp.sqrt(v + 1e-5) * g_ref[...] + b_ref[...]
    tok_ref[...] = tok
    tokb = tok.astype(_bf16)
    tokb_ref[...] = tokb
    lg_ref[...] = _dot(tokb, wg_ref[...].astype(_bf16))


def _route_kern(lg_ref, wa_ref, wb_ref, pa_ref, pb_ref, ext_ref, used_ref,
                aux_ref):
    lg = lg_ref[...]
    mx = jnp.max(lg, axis=-1, keepdims=True)
    exl = jnp.exp(lg - mx)
    p = exl / jnp.sum(exl, axis=-1, keepdims=True)
    iota = jax.lax.broadcasted_iota(jnp.int32, (S, E), 1)
    m1 = jnp.max(p, axis=-1, keepdims=True)
    i1 = jnp.min(jnp.where(p == m1, iota, E), axis=-1, keepdims=True)
    pm = jnp.where(iota == i1, -1.0, p)
    m2 = jnp.max(pm, axis=-1, keepdims=True)
    i2 = jnp.min(jnp.where(pm == m2, iota, E), axis=-1, keepdims=True)
    wsum = m1 + m2
    wa_ref[...] = m1 / wsum
    wb_ref[...] = m2 / wsum

    ohA = (iota == i1).astype(_f32)
    ohB = (iota == i2).astype(_f32)
    oh = ohA + ohB

    r = jax.lax.broadcasted_iota(jnp.int32, (CH, CH), 0)
    c = jax.lax.broadcasted_iota(jnp.int32, (CH, CH), 1)
    tril = (r > c).astype(_bf16)
    carry = jnp.zeros((1, E), _f32)
    parts = []
    for ci in range(S // CH):
        blk = oh[ci * CH:(ci + 1) * CH]
        parts.append(_dot(tril, blk.astype(_bf16)) + carry)
        carry = carry + jnp.sum(blk, axis=0, keepdims=True)
    C = jnp.concatenate(parts, axis=0)
    counts = carry

    padded = jnp.floor((counts + (TEX - 1)) * (1.0 / TEX)) * TEX
    er = jax.lax.broadcasted_iota(jnp.int32, (E, E), 0)
    ec = jax.lax.broadcasted_iota(jnp.int32, (E, E), 1)
    off = _dot(padded.astype(_bf16), (er < ec).astype(_bf16))
    end_tile = (off + padded) * (1.0 / TEX)

    posA = jnp.sum(ohA * (off + C), axis=-1, keepdims=True)
    posB = jnp.sum(ohB * (off + C), axis=-1, keepdims=True)
    sub = jax.lax.broadcasted_iota(jnp.int32, (S, NSUB), 1)
    pa_ref[...] = posA.astype(jnp.int32) * NSUB + sub
    pb_ref[...] = posB.astype(jnp.int32) * NSUB + sub

    gi = jax.lax.broadcasted_iota(jnp.int32, (1, GMAX), 1).astype(_f32)
    ext = jnp.zeros((1, GMAX), _f32)
    for e in range(E):
        ext = ext + (gi >= end_tile[:, e:e + 1]).astype(_f32)
    ext_ref[...] = jnp.minimum(ext, E - 1).astype(jnp.int32)
    used_ref[...] = end_tile[:, E - 1:E].astype(jnp.int32)

    sel = jnp.logical_or(iota == i1, iota == i2)
    fi = jnp.sum(sel.astype(_f32), axis=0, keepdims=True) / (S * K)
    Pi = jnp.mean(p, axis=0, keepdims=True)
    aux_ref[...] = 0.01 * E * jnp.sum(fi * Pi, axis=-1, keepdims=True)


def _vmesh():
    return plsc.VectorSubcoreMesh(core_axis_name="c", subcore_axis_name="s")


def _sc_dispatch(tok8, pa2, pb2):
    @pl.kernel(out_type=jax.ShapeDtypeStruct((P * NSUB, LW), _f32),
               mesh=_vmesh(), scratch_types=[])
    def k(tok_hbm, ia_hbm, ib_hbm, o_hbm):
        def scat(x_vmem, i_vmem):
            pltpu.sync_copy(x_vmem, o_hbm.at[i_vmem.at[0]])

        for idx_hbm in (ia_hbm, ib_hbm):
            pltpu.emit_pipeline(
                scat,
                grid=(S * NSUB // SCW,),
                in_specs=[pl.BlockSpec((SCW, LW), index_map=lambda i: (i, 0)),
                          pl.BlockSpec((1, SCW), index_map=lambda i: (0, i))],
                out_specs=[],
                core_axis_name=("c", "s"),
                dimension_semantics=(pltpu.PARALLEL,),
            )(tok_hbm, idx_hbm)

    return k(tok8, pa2, pb2)


def _sc_combine(eo8, pa2, pb2):
    @pl.kernel(out_type=(jax.ShapeDtypeStruct((S * NSUB, LW), _f32),
                         jax.ShapeDtypeStruct((S * NSUB, LW), _f32)),
               mesh=_vmesh(), scratch_types=[])
    def k(eo_hbm, ia_hbm, ib_hbm, ga_hbm, gb_hbm):
        def gath(i_vmem, o_vmem):
            pltpu.sync_copy(eo_hbm.at[i_vmem.at[0]], o_vmem)

        for idx_hbm, out_hbm in ((ia_hbm, ga_hbm), (ib_hbm, gb_hbm)):
            pltpu.emit_pipeline(
                gath,
                grid=(S * NSUB // SCW,),
                in_specs=[pl.BlockSpec((1, SCW), index_map=lambda i: (0, i))],
                out_specs=[pl.BlockSpec((SCW, LW), index_map=lambda i: (i, 0))],
                core_axis_name=("c", "s"),
                dimension_semantics=(pltpu.PARALLEL,),
            )(idx_hbm, out_hbm)

    return k(eo8, pa2, pb2)


def _gmm_kern(ext_ref, used_ref, x_ref, we1_ref, be1_ref, we2_ref, be2_ref,
              o_ref):
    g = pl.program_id(0)

    @pl.when(g < used_ref[0])
    def _():
        xb = x_ref[...].astype(_bf16)
        eh = jax.nn.gelu(_dot(xb, we1_ref[0]) + be1_ref[0])
        o_ref[...] = _dot(eh.astype(_bf16), we2_ref[0]) + be2_ref[0]


def _shared_kern(tok_ref, bs2_ref, ws1_ref, bs1_ref, ws2_ref, o_ref):
    hb = pl.program_id(0)
    t = pl.program_id(1)
    rows = pl.ds(t * BT, BT)
    tokb = tok_ref[rows, :]
    eh = jax.nn.gelu(_dot(tokb, ws1_ref[...]) + bs1_ref[...])
    acc = _dot(eh.astype(_bf16), ws2_ref[...])

    @pl.when(hb == 0)
    def _():
        o_ref[rows, :] = acc + bs2_ref[...]

    @pl.when(hb != 0)
    def _():
        o_ref[rows, :] += acc


def _comb_kern(x1_ref, sh_ref, wa_ref, wb_ref, ga_ref, gb_ref, o_ref):
    routed = wa_ref[...] * ga_ref[...] + wb_ref[...] * gb_ref[...]
    o_ref[...] = x1_ref[...] + sh_ref[...] + routed


def kernel(x, ln1_g, ln1_b, Wqkv, bqkv, Wo, bo, ln2_g, ln2_b, Wg,
           We1, be1, We2, be2, Ws1, bs1, Ws2, bs2):
    x2 = x.reshape(S, D)
    row1 = lambda a: a.reshape(1, -1)

    qkv = pl.pallas_call(
        _qkv_kern,
        grid=(NT,),
        in_specs=[
            pl.BlockSpec((BT, D), lambda i: (i, 0)),
            pl.BlockSpec((1, D), lambda i: (0, 0)),
            pl.BlockSpec((1, D), lambda i: (0, 0)),
            pl.BlockSpec((D, 3 * D), lambda i: (0, 0)),
            pl.BlockSpec((1, 3 * D), lambda i: (0, 0)),
        ],
        out_specs=pl.BlockSpec((BT, 3 * D), lambda i: (i, 0)),
        out_shape=jax.ShapeDtypeStruct((S, 3 * D), _f32),
    )(x2, row1(ln1_g), row1(ln1_b), Wqkv.astype(_bf16), row1(bqkv))

    q3 = qkv[:, :D].reshape(S, H, DH).transpose(1, 0, 2)
    k3 = qkv[:, D:2 * D].reshape(S, H, DH).transpose(1, 0, 2)
    v3 = qkv[:, 2 * D:].reshape(S, H, DH).transpose(1, 0, 2)

    ctx3 = pl.pallas_call(
        _attn_kern,
        grid=(H, NQ),
        in_specs=[
            pl.BlockSpec((1, BQ, DH), lambda h, t: (h, t, 0)),
            pl.BlockSpec((1, S, DH), lambda h, t: (h, 0, 0)),
            pl.BlockSpec((1, S, DH), lambda h, t: (h, 0, 0)),
        ],
        out_specs=pl.BlockSpec((1, BQ, DH), lambda h, t: (h, t, 0)),
        out_shape=jax.ShapeDtypeStruct((H, S, DH), _f32),
        scratch_shapes=[pltpu.VMEM((BQ, S), _f32)],
    )(q3, k3, v3)
    ctx = ctx3.transpose(1, 0, 2).reshape(S, D)

    x1, tok, tokb, logits = pl.pallas_call(
        _post_kern,
        grid=(NT,),
        in_specs=[
            pl.BlockSpec((BT, D), lambda i: (i, 0)),
            pl.BlockSpec((D, D), lambda i: (0, 0)),
            pl.BlockSpec((1, D), lambda i: (0, 0)),
            pl.BlockSpec((BT, D), lambda i: (i, 0)),
            pl.BlockSpec((1, D), lambda i: (0, 0)),
            pl.BlockSpec((1, D), lambda i: (0, 0)),
            pl.BlockSpec((D, E), lambda i: (0, 0)),
        ],
        out_specs=[
            pl.BlockSpec((BT, D), lambda i: (i, 0)),
            pl.BlockSpec((BT, D), lambda i: (i, 0)),
            pl.BlockSpec((BT, D), lambda i: (i, 0)),
            pl.BlockSpec((BT, E), lambda i: (i, 0)),
        ],
        out_shape=[
            jax.ShapeDtypeStruct((S, D), _f32),
            jax.ShapeDtypeStruct((S, D), _f32),
            jax.ShapeDtypeStruct((S, D), _bf16),
            jax.ShapeDtypeStruct((S, E), _f32),
        ],
    )(ctx, Wo.astype(_bf16), row1(bo), x2, row1(ln2_g), row1(ln2_b), Wg)

    wa, wb, pa, pb, ext, used, aux = pl.pallas_call(
        _route_kern,
        grid=(1,),
        in_specs=[pl.BlockSpec((S, E), lambda i: (0, 0))],
        out_specs=[
            pl.BlockSpec((S, 1), lambda i: (0, 0)),
            pl.BlockSpec((S, 1), lambda i: (0, 0)),
            pl.BlockSpec((S, NSUB), lambda i: (0, 0)),
            pl.BlockSpec((S, NSUB), lambda i: (0, 0)),
            pl.BlockSpec((1, GMAX), lambda i: (0, 0)),
            pl.BlockSpec((1, 1), lambda i: (0, 0)),
            pl.BlockSpec((1, 1), lambda i: (0, 0)),
        ],
        out_shape=[
            jax.ShapeDtypeStruct((S, 1), _f32),
            jax.ShapeDtypeStruct((S, 1), _f32),
            jax.ShapeDtypeStruct((S, NSUB), jnp.int32),
            jax.ShapeDtypeStruct((S, NSUB), jnp.int32),
            jax.ShapeDtypeStruct((1, GMAX), jnp.int32),
            jax.ShapeDtypeStruct((1, 1), jnp.int32),
            jax.ShapeDtypeStruct((1, 1), _f32),
        ],
    )(logits)

    pa2 = pa.reshape(1, S * NSUB)
    pb2 = pb.reshape(1, S * NSUB)

    xs = _sc_dispatch(tok.reshape(S * NSUB, LW), pa2, pb2).reshape(P, D)

    eo = pl.pallas_call(
        _gmm_kern,
        grid_spec=pltpu.PrefetchScalarGridSpec(
            num_scalar_prefetch=2,
            grid=(GMAX,),
            in_specs=[
                pl.BlockSpec((TEX, D), lambda g, ext, used: (g, 0)),
                pl.BlockSpec((1, D, HID), lambda g, ext, used: (ext[g], 0, 0)),
                pl.BlockSpec((1, 1, HID), lambda g, ext, used: (ext[g], 0, 0)),
                pl.BlockSpec((1, HID, D), lambda g, ext, used: (ext[g], 0, 0)),
                pl.BlockSpec((1, 1, D), lambda g, ext, used: (ext[g], 0, 0)),
            ],
            out_specs=pl.BlockSpec((TEX, D), lambda g, ext, used: (g, 0)),
        ),
        out_shape=jax.ShapeDtypeStruct((P, D), _f32),
    )(ext.reshape(GMAX), used.reshape(1), xs, We1.astype(_bf16),
      be1.reshape(E, 1, HID), We2.astype(_bf16), be2.reshape(E, 1, D))

    ga, gb = _sc_combine(eo.reshape(P * NSUB, LW), pa2, pb2)
    ga = ga.reshape(S, D)
    gb = gb.reshape(S, D)

    shared = pl.pallas_call(
        _shared_kern,
        grid=(NHB, NT),
        in_specs=[
            pl.BlockSpec((S, D), lambda hb, t: (0, 0)),
            pl.BlockSpec((1, D), lambda hb, t: (0, 0)),
            pl.BlockSpec((D, HB), lambda hb, t: (0, hb)),
            pl.BlockSpec((1, HB), lambda hb, t: (0, hb)),
            pl.BlockSpec((HB, D), lambda hb, t: (hb, 0)),
        ],
        out_specs=pl.BlockSpec((S, D), lambda hb, t: (0, 0)),
        out_shape=jax.ShapeDtypeStruct((S, D), _f32),
    )(tokb, row1(bs2), Ws1.astype(_bf16), row1(bs1), Ws2.astype(_bf16))

    out = pl.pallas_call(
        _comb_kern,
        grid=(NT,),
        in_specs=[
            pl.BlockSpec((BT, D), lambda i: (i, 0)),
            pl.BlockSpec((BT, D), lambda i: (i, 0)),
            pl.BlockSpec((BT, 1), lambda i: (i, 0)),
            pl.BlockSpec((BT, 1), lambda i: (i, 0)),
            pl.BlockSpec((BT, D), lambda i: (i, 0)),
            pl.BlockSpec((BT, D), lambda i: (i, 0)),
        ],
        out_specs=pl.BlockSpec((BT, D), lambda i: (i, 0)),
        out_shape=jax.ShapeDtypeStruct((S, D), _f32),
    )(x1, shared, wa, wb, ga, gb)

    return (aux[0, 0], out.reshape(B, S, D))


# speedup vs baseline: 1.2838x; 1.0481x over previous
import jax
import jax.numpy as jnp
from jax.experimental import pallas as pl
from jax.experimental.pallas import tpu as pltpu
from jax.experimental.pallas import tpu_sc as plsc

B, S, D, H = 1, 2048, 1024, 16
E, K, HID = 4, 2, 4096
DH = D // H
BT = 256
NT = S // BT
BQ = 1024
NQ = S // BQ
HB = 512
NHB = HID // HB
TEX = 512
GMAX = 11
P = GMAX * TEX
CH = 512
LW = 128
NSUB = D // LW
SCW = 128

_f32 = jnp.float32
_bf16 = jnp.bfloat16


def _dot(a, b, trans_b=False, prec=None):
    dims = (((1,), (1 if trans_b else 0,)), ((), ()))
    return jax.lax.dot_general(a, b, dims, preferred_element_type=_f32,
                               precision=prec)


def _qkv_kern(x_ref, g_ref, b_ref, w_ref, bias_ref, o_ref):
    x = x_ref[...]
    m = jnp.mean(x, axis=-1, keepdims=True)
    v = jnp.mean((x - m) ** 2, axis=-1, keepdims=True)
    h = (x - m) / jnp.sqrt(v + 1e-5) * g_ref[...] + b_ref[...]
    o_ref[...] = _dot(h.astype(_bf16), w_ref[...]) + bias_ref[...]


def _attn_kern(q_ref, k_ref, v_ref, o_ref, s_scr):
    t = pl.program_id(1)
    q = q_ref[0].astype(_bf16)

    def fill(kb, c):
        @pl.when(kb <= t)
        def _():
            kblk = k_ref[0, pl.ds(kb * BQ, BQ), :].astype(_bf16)
            s = _dot(q, kblk, trans_b=True) * (1.0 / 8.0)
            row = t * BQ + jax.lax.broadcasted_iota(jnp.int32, (BQ, BQ), 0)
            col = kb * BQ + jax.lax.broadcasted_iota(jnp.int32, (BQ, BQ), 1)
            s_scr[:, pl.ds(kb * BQ, BQ)] = jnp.where(col <= row, s, -1e9)

        @pl.when(kb > t)
        def _():
            s_scr[:, pl.ds(kb * BQ, BQ)] = jnp.full((BQ, BQ), -1e9, _f32)

        return c

    jax.lax.fori_loop(0, NQ, fill, 0)
    s = s_scr[...]
    m = jnp.max(s, axis=-1, keepdims=True)
    e = jnp.exp(s - m)
    p = e / jnp.sum(e, axis=-1, keepdims=True)
    o_ref[0] = _dot(p.astype(_bf16), v_ref[0].astype(_bf16))


def _post_kern(ctx_ref, wo_ref, bo_ref, x_ref, g_ref, b_ref, wg_ref,
               x1_ref, tok_ref, tokb_ref, lg_ref):
    ao = _dot(ctx_ref[...].astype(_bf16), wo_ref[...]) + bo_ref[...]
    x1 = x_ref[...] + ao
    x1_ref[...] = x1
    m = jnp.mean(x1, axis=-1, keepdims=True)
    v = jnp.mean((x1 - m) ** 2, axis=-1, keepdims=True)
    tok = (x1 - m) / jnp.sqrt(v + 1e-5) * g_ref[...] + b_ref[...]
    tok_ref[...] = tok
    tokb = tok.astype(_bf16)
    tokb_ref[...] = tokb
    lg_ref[...] = _dot(tokb, wg_ref[...].astype(_bf16))


def _route_kern(lg_ref, wa_ref, wb_ref, pa_ref, pb_ref, ext_ref, used_ref,
                aux_ref):
    lg = lg_ref[...]
    mx = jnp.max(lg, axis=-1, keepdims=True)
    exl = jnp.exp(lg - mx)
    p = exl / jnp.sum(exl, axis=-1, keepdims=True)
    iota = jax.lax.broadcasted_iota(jnp.int32, (S, E), 1)
    m1 = jnp.max(p, axis=-1, keepdims=True)
    i1 = jnp.min(jnp.where(p == m1, iota, E), axis=-1, keepdims=True)
    pm = jnp.where(iota == i1, -1.0, p)
    m2 = jnp.max(pm, axis=-1, keepdims=True)
    i2 = jnp.min(jnp.where(pm == m2, iota, E), axis=-1, keepdims=True)
    wsum = m1 + m2
    wa_ref[...] = m1 / wsum
    wb_ref[...] = m2 / wsum

    ohA = (iota == i1).astype(_f32)
    ohB = (iota == i2).astype(_f32)
    oh = ohA + ohB

    r = jax.lax.broadcasted_iota(jnp.int32, (CH, CH), 0)
    c = jax.lax.broadcasted_iota(jnp.int32, (CH, CH), 1)
    tril = (r > c).astype(_bf16)
    carry = jnp.zeros((1, E), _f32)
    parts = []
    for ci in range(S // CH):
        blk = oh[ci * CH:(ci + 1) * CH]
        parts.append(_dot(tril, blk.astype(_bf16)) + carry)
        carry = carry + jnp.sum(blk, axis=0, keepdims=True)
    C = jnp.concatenate(parts, axis=0)
    counts = carry

    padded = jnp.floor((counts + (TEX - 1)) * (1.0 / TEX)) * TEX
    er = jax.lax.broadcasted_iota(jnp.int32, (E, E), 0)
    ec = jax.lax.broadcasted_iota(jnp.int32, (E, E), 1)
    off = _dot(padded.astype(_bf16), (er < ec).astype(_bf16))
    end_tile = (off + padded) * (1.0 / TEX)

    posA = jnp.sum(ohA * (off + C), axis=-1, keepdims=True)
    posB = jnp.sum(ohB * (off + C), axis=-1, keepdims=True)
    sub = jax.lax.broadcasted_iota(jnp.int32, (S, NSUB), 1)
    pa_ref[...] = posA.astype(jnp.int32) * NSUB + sub
    pb_ref[...] = posB.astype(jnp.int32) * NSUB + sub

    gi = jax.lax.broadcasted_iota(jnp.int32, (1, GMAX), 1).astype(_f32)
    ext = jnp.zeros((1, GMAX), _f32)
    for e in range(E):
        ext = ext + (gi >= end_tile[:, e:e + 1]).astype(_f32)
    ext_ref[...] = jnp.minimum(ext, E - 1).astype(jnp.int32)
    used_ref[...] = end_tile[:, E - 1:E].astype(jnp.int32)

    sel = jnp.logical_or(iota == i1, iota == i2)
    fi = jnp.sum(sel.astype(_f32), axis=0, keepdims=True) / (S * K)
    Pi = jnp.mean(p, axis=0, keepdims=True)
    aux_ref[...] = 0.01 * E * jnp.sum(fi * Pi, axis=-1, keepdims=True)


def _vmesh():
    return plsc.VectorSubcoreMesh(core_axis_name="c", subcore_axis_name="s")


def _sc_dispatch(tok8, pa2, pb2):
    @pl.kernel(out_type=jax.ShapeDtypeStruct((P * NSUB, LW), _f32),
               mesh=_vmesh(), scratch_types=[])
    def k(tok_hbm, ia_hbm, ib_hbm, o_hbm):
        def scat(x_vmem, i_vmem):
            pltpu.sync_copy(x_vmem, o_hbm.at[i_vmem.at[0]])

        for idx_hbm in (ia_hbm, ib_hbm):
            pltpu.emit_pipeline(
                scat,
                grid=(S * NSUB // SCW,),
                in_specs=[pl.BlockSpec((SCW, LW), index_map=lambda i: (i, 0)),
                          pl.BlockSpec((1, SCW), index_map=lambda i: (0, i))],
                out_specs=[],
                core_axis_name=("c", "s"),
                dimension_semantics=(pltpu.PARALLEL,),
            )(tok_hbm, idx_hbm)

    return k(tok8, pa2, pb2)


def _sc_combine(eo8, pa2, pb2):
    @pl.kernel(out_type=(jax.ShapeDtypeStruct((S * NSUB, LW), _f32),
                         jax.ShapeDtypeStruct((S * NSUB, LW), _f32)),
               mesh=_vmesh(), scratch_types=[])
    def k(eo_hbm, ia_hbm, ib_hbm, ga_hbm, gb_hbm):
        def gath(i_vmem, o_vmem):
            pltpu.sync_copy(eo_hbm.at[i_vmem.at[0]], o_vmem)

        for idx_hbm, out_hbm in ((ia_hbm, ga_hbm), (ib_hbm, gb_hbm)):
            pltpu.emit_pipeline(
                gath,
                grid=(S * NSUB // SCW,),
                in_specs=[pl.BlockSpec((1, SCW), index_map=lambda i: (0, i))],
                out_specs=[pl.BlockSpec((SCW, LW), index_map=lambda i: (i, 0))],
                core_axis_name=("c", "s"),
                dimension_semantics=(pltpu.PARALLEL,),
            )(idx_hbm, out_hbm)

    return k(eo8, pa2, pb2)


def _gmm_kern(ext_ref, used_ref, x_ref, we1_ref, be1_ref, we2_ref, be2_ref,
              o_ref):
    g = pl.program_id(0)

    @pl.when(g < used_ref[0])
    def _():
        xb = x_ref[...].astype(_bf16)
        eh = jax.nn.gelu(_dot(xb, we1_ref[0]) + be1_ref[0])
        o_ref[...] = _dot(eh.astype(_bf16), we2_ref[0]) + be2_ref[0]


def _shared_kern(tok_ref, bs2_ref, ws1_ref, bs1_ref, ws2_ref, o_ref):
    hb = pl.program_id(0)
    t = pl.program_id(1)
    rows = pl.ds(t * BT, BT)
    tokb = tok_ref[rows, :]
    eh = jax.nn.gelu(_dot(tokb, ws1_ref[...]) + bs1_ref[...])
    acc = _dot(eh.astype(_bf16), ws2_ref[...])

    @pl.when(hb == 0)
    def _():
        o_ref[rows, :] = acc + bs2_ref[...]

    @pl.when(hb != 0)
    def _():
        o_ref[rows, :] += acc


def _comb_kern(x1_ref, sh_ref, wa_ref, wb_ref, ga_ref, gb_ref, o_ref):
    routed = wa_ref[...] * ga_ref[...] + wb_ref[...] * gb_ref[...]
    o_ref[...] = x1_ref[...] + sh_ref[...] + routed


def kernel(x, ln1_g, ln1_b, Wqkv, bqkv, Wo, bo, ln2_g, ln2_b, Wg,
           We1, be1, We2, be2, Ws1, bs1, Ws2, bs2):
    x2 = x.reshape(S, D)
    row1 = lambda a: a.reshape(1, -1)

    qkv = pl.pallas_call(
        _qkv_kern,
        grid=(NT,),
        in_specs=[
            pl.BlockSpec((BT, D), lambda i: (i, 0)),
            pl.BlockSpec((1, D), lambda i: (0, 0)),
            pl.BlockSpec((1, D), lambda i: (0, 0)),
            pl.BlockSpec((D, 3 * D), lambda i: (0, 0)),
            pl.BlockSpec((1, 3 * D), lambda i: (0, 0)),
        ],
        out_specs=pl.BlockSpec((BT, 3 * D), lambda i: (i, 0)),
        out_shape=jax.ShapeDtypeStruct((S, 3 * D), _f32),
    )(x2, row1(ln1_g), row1(ln1_b), Wqkv.astype(_bf16), row1(bqkv))

    q3 = qkv[:, :D].reshape(S, H, DH).transpose(1, 0, 2)
    k3 = qkv[:, D:2 * D].reshape(S, H, DH).transpose(1, 0, 2)
    v3 = qkv[:, 2 * D:].reshape(S, H, DH).transpose(1, 0, 2)

    ctx3 = pl.pallas_call(
        _attn_kern,
        grid=(H, NQ),
        in_specs=[
            pl.BlockSpec((1, BQ, DH), lambda h, t: (h, t, 0)),
            pl.BlockSpec((1, S, DH), lambda h, t: (h, 0, 0)),
            pl.BlockSpec((1, S, DH), lambda h, t: (h, 0, 0)),
        ],
        out_specs=pl.BlockSpec((1, BQ, DH), lambda h, t: (h, t, 0)),
        out_shape=jax.ShapeDtypeStruct((H, S, DH), _f32),
        scratch_shapes=[pltpu.VMEM((BQ, S), _f32)],
    )(q3, k3, v3)
    ctx = ctx3.transpose(1, 0, 2).reshape(S, D)

    x1, tok, tokb, logits = pl.pallas_call(
        _post_kern,
        grid=(NT,),
        in_specs=[
            pl.BlockSpec((BT, D), lambda i: (i, 0)),
            pl.BlockSpec((D, D), lambda i: (0, 0)),
            pl.BlockSpec((1, D), lambda i: (0, 0)),
            pl.BlockSpec((BT, D), lambda i: (i, 0)),
            pl.BlockSpec((1, D), lambda i: (0, 0)),
            pl.BlockSpec((1, D), lambda i: (0, 0)),
            pl.BlockSpec((D, E), lambda i: (0, 0)),
        ],
        out_specs=[
            pl.BlockSpec((BT, D), lambda i: (i, 0)),
            pl.BlockSpec((BT, D), lambda i: (i, 0)),
            pl.BlockSpec((BT, D), lambda i: (i, 0)),
            pl.BlockSpec((BT, E), lambda i: (i, 0)),
        ],
        out_shape=[
            jax.ShapeDtypeStruct((S, D), _f32),
            jax.ShapeDtypeStruct((S, D), _f32),
            jax.ShapeDtypeStruct((S, D), _bf16),
            jax.ShapeDtypeStruct((S, E), _f32),
        ],
    )(ctx, Wo.astype(_bf16), row1(bo), x2, row1(ln2_g), row1(ln2_b), Wg)

    wa, wb, pa, pb, ext, used, aux = pl.pallas_call(
        _route_kern,
        grid=(1,),
        in_specs=[pl.BlockSpec((S, E), lambda i: (0, 0))],
        out_specs=[
            pl.BlockSpec((S, 1), lambda i: (0, 0)),
            pl.BlockSpec((S, 1), lambda i: (0, 0)),
            pl.BlockSpec((S, NSUB), lambda i: (0, 0)),
            pl.BlockSpec((S, NSUB), lambda i: (0, 0)),
            pl.BlockSpec((1, GMAX), lambda i: (0, 0)),
            pl.BlockSpec((1, 1), lambda i: (0, 0)),
            pl.BlockSpec((1, 1), lambda i: (0, 0)),
        ],
        out_shape=[
            jax.ShapeDtypeStruct((S, 1), _f32),
            jax.ShapeDtypeStruct((S, 1), _f32),
            jax.ShapeDtypeStruct((S, NSUB), jnp.int32),
            jax.ShapeDtypeStruct((S, NSUB), jnp.int32),
            jax.ShapeDtypeStruct((1, GMAX), jnp.int32),
            jax.ShapeDtypeStruct((1, 1), jnp.int32),
            jax.ShapeDtypeStruct((1, 1), _f32),
        ],
    )(logits)

    pa2 = pa.reshape(1, S * NSUB)
    pb2 = pb.reshape(1, S * NSUB)

    xs = _sc_dispatch(tok.reshape(S * NSUB, LW), pa2, pb2).reshape(P, D)

    eo = pl.pallas_call(
        _gmm_kern,
        grid_spec=pltpu.PrefetchScalarGridSpec(
            num_scalar_prefetch=2,
            grid=(GMAX,),
            in_specs=[
                pl.BlockSpec((TEX, D), lambda g, ext, used: (g, 0)),
                pl.BlockSpec((1, D, HID), lambda g, ext, used: (ext[g], 0, 0)),
                pl.BlockSpec((1, 1, HID), lambda g, ext, used: (ext[g], 0, 0)),
                pl.BlockSpec((1, HID, D), lambda g, ext, used: (ext[g], 0, 0)),
                pl.BlockSpec((1, 1, D), lambda g, ext, used: (ext[g], 0, 0)),
            ],
            out_specs=pl.BlockSpec((TEX, D), lambda g, ext, used: (g, 0)),
        ),
        out_shape=jax.ShapeDtypeStruct((P, D), _f32),
    )(ext.reshape(GMAX), used.reshape(1), xs, We1.astype(_bf16),
      be1.reshape(E, 1, HID), We2.astype(_bf16), be2.reshape(E, 1, D))

    ga, gb = _sc_combine(eo.reshape(P * NSUB, LW), pa2, pb2)
    ga = ga.reshape(S, D)
    gb = gb.reshape(S, D)

    shared = pl.pallas_call(
        _shared_kern,
        grid=(NHB, NT),
        in_specs=[
            pl.BlockSpec((S, D), lambda hb, t: (0, 0)),
            pl.BlockSpec((1, D), lambda hb, t: (0, 0)),
            pl.BlockSpec((D, HB), lambda hb, t: (0, hb)),
            pl.BlockSpec((1, HB), lambda hb, t: (0, hb)),
            pl.BlockSpec((HB, D), lambda hb, t: (hb, 0)),
        ],
        out_specs=pl.BlockSpec((S, D), lambda hb, t: (0, 0)),
        out_shape=jax.ShapeDtypeStruct((S, D), _f32),
    )(tokb, row1(bs2), Ws1.astype(_bf16), row1(bs1), Ws2.astype(_bf16))

    out = pl.pallas_call(
        _comb_kern,
        grid=(NT,),
        in_specs=[
            pl.BlockSpec((BT, D), lambda i: (i, 0)),
            pl.BlockSpec((BT, D), lambda i: (i, 0)),
            pl.BlockSpec((BT, 1), lambda i: (i, 0)),
            pl.BlockSpec((BT, 1), lambda i: (i, 0)),
            pl.BlockSpec((BT, D), lambda i: (i, 0)),
            pl.BlockSpec((BT, D), lambda i: (i, 0)),
        ],
        out_specs=pl.BlockSpec((BT, D), lambda i: (i, 0)),
        out_shape=jax.ShapeDtypeStruct((S, D), _f32),
    )(x1, shared, wa, wb, ga, gb)

    return (aux[0, 0], out.reshape(B, S, D))
